# baseline (speedup 1.0000x reference)
import functools

import jax
import jax.numpy as jnp
from jax import lax
from jax.experimental import pallas as pl
from jax.experimental.pallas import tpu as pltpu
from jax.experimental.pallas import tpu_sc as plsc

N, E, D, DE = 10000, 320000, 128, 16
IDIM = 2 * D + DE

C = 10
EC = E // C

NC, NS = 2, 16
NW = NC * NS
E_PER = EC // NW
CB = 40
ITERS = E_PER // CB

BE = 3200
NBLK = EC // BE
BE2 = 16000
NBLK2 = EC // BE2


def _gather_body(coff, idx_hbm, table_hbm, out_i_hbm, out_j_hbm,
                 idx_i_v, idx_j_v, rows_i, rows_j, gsem_i, gsem_j,
                 wsem_i, wsem_j):
    wid = lax.axis_index("s") * NC + lax.axis_index("c")
    base0 = wid * E_PER

    pltpu.sync_copy(idx_hbm.at[pl.ds(coff + base0, E_PER)], idx_i_v)
    pltpu.sync_copy(idx_hbm.at[pl.ds(E + coff + base0, E_PER)], idx_j_v)

    def gather(t):
        b = t % 2
        gi = pltpu.async_copy(table_hbm.at[idx_i_v.at[pl.ds(t * CB, CB)]],
                              rows_i[b], gsem_i[b])
        gj = pltpu.async_copy(table_hbm.at[idx_j_v.at[pl.ds(t * CB, CB)]],
                              rows_j[b], gsem_j[b])
        return gi, gj

    def write(t):
        b = t % 2
        base = base0 + t * CB
        wi = pltpu.async_copy(rows_i[b], out_i_hbm.at[pl.ds(base, CB)],
                              wsem_i[b])
        wj = pltpu.async_copy(rows_j[b], out_j_hbm.at[pl.ds(base, CB)],
                              wsem_j[b])
        return wi, wj

    gd = {0: gather(0)}
    wd = {}
    for t in range(ITERS):
        if t + 1 < ITERS:
            if t - 1 >= 0:
                for c in wd.pop(t - 1):
                    c.wait()
            gd[t + 1] = gather(t + 1)
        for c in gd.pop(t):
            c.wait()
        wd[t] = write(t)
    for t in (ITERS - 2, ITERS - 1):
        for c in wd.pop(t, ()):
            c.wait()


def _sc_gather(table, idx_flat, coff):
    mesh = plsc.VectorSubcoreMesh(core_axis_name="c", subcore_axis_name="s")
    k = pl.kernel(
        functools.partial(_gather_body, coff),
        out_type=[
            jax.ShapeDtypeStruct((EC, D), jnp.float32),
            jax.ShapeDtypeStruct((EC, D), jnp.float32),
        ],
        mesh=mesh,
        scratch_types=[
            pltpu.VMEM((E_PER,), jnp.int32),
            pltpu.VMEM((E_PER,), jnp.int32),
            [pltpu.VMEM((CB, D), jnp.float32) for _ in range(2)],
            [pltpu.VMEM((CB, D), jnp.float32) for _ in range(2)],
            [pltpu.SemaphoreType.DMA for _ in range(2)],
            [pltpu.SemaphoreType.DMA for _ in range(2)],
            [pltpu.SemaphoreType.DMA for _ in range(2)],
            [pltpu.SemaphoreType.DMA for _ in range(2)],
        ],
    )
    return k(idx_flat, table)


def _pass1_body(gi_ref, gj_ref, ef_ref, w1a_ref, w1b_ref, w1c_ref,
                z_ref, stats_ref):
    step = pl.program_id(0)
    gi = gi_ref[...]
    gj = gj_ref[...]
    s = (gi + gj).astype(jnp.bfloat16)
    d = jnp.abs(gi - gj).astype(jnp.bfloat16)
    eft = ef_ref[...].astype(jnp.bfloat16)
    z = jnp.dot(s, w1a_ref[...], preferred_element_type=jnp.float32)
    z += jnp.dot(d, w1b_ref[...], preferred_element_type=jnp.float32)
    z += lax.dot_general(eft, w1c_ref[...], (((0,), (0,)), ((), ())),
                         preferred_element_type=jnp.float32)
    z_ref[...] = z.astype(z_ref.dtype)

    sums = jnp.sum(z, axis=0)
    sqs = jnp.sum(z * z, axis=0)
    acc = jnp.concatenate(
        [sums[None, :], sqs[None, :], jnp.zeros((6, IDIM), jnp.float32)], axis=0)

    @pl.when(step == 0)
    def _():
        stats_ref[...] = jnp.zeros_like(stats_ref)

    stats_ref[...] += acc


def _pass2_body_aliased(z_ref, stats_ref, gamma_ref, beta_ref, w2t_ref,
                        b2_ref, buf_ref, out_ref):
    del buf_ref
    _pass2_body(z_ref, stats_ref, gamma_ref, beta_ref, w2t_ref, b2_ref,
                out_ref)


def _pass2_body(z_ref, stats_ref, gamma_ref, beta_ref, w2t_ref, b2_ref,
                out_ref):
    st = stats_ref[...]
    tot = (st[0:8, :] + st[8:16, :] + st[16:24, :] + st[24:32, :]
           + st[32:40, :])
    mean = tot[0:1, :] / E
    var = tot[1:2, :] / E - mean * mean
    a = gamma_ref[...] * lax.rsqrt(var + 1e-5)
    c = beta_ref[...] - mean * a
    y = jnp.maximum(z_ref[...].astype(jnp.float32) * a + c, 0.0)
    out_ref[...] = (
        lax.dot_general(w2t_ref[...], y, (((0,), (1,)), ((), ())),
                        preferred_element_type=jnp.float32)
        + b2_ref[...])


def kernel(node_feats, edge_feats, edge_index, W1, b1, gamma, beta, W2, b2):
    del b1
    idx_flat = edge_index.reshape(2 * E)
    bf = jnp.bfloat16

    w1a = W1[:, :D].T.astype(bf)
    w1b = W1[:, D:2 * D].T.astype(bf)
    w1c = W1[:, 2 * D:].T.astype(bf)
    w2t = W2.T
    gamma2 = gamma.reshape(1, IDIM)
    beta2 = beta.reshape(1, IDIM)
    b22 = b2.reshape(DE, 1)
    ef_t = edge_feats.T

    def make_pass1(c):
        return pl.pallas_call(
            _pass1_body,
            grid=(NBLK,),
            in_specs=[
                pl.BlockSpec((BE, D), lambda i: (i, 0)),
                pl.BlockSpec((BE, D), lambda i: (i, 0)),
                pl.BlockSpec((DE, BE), lambda i, c=c: (0, i + c * NBLK)),
                pl.BlockSpec((D, IDIM), lambda i: (0, 0)),
                pl.BlockSpec((D, IDIM), lambda i: (0, 0)),
                pl.BlockSpec((DE, IDIM), lambda i: (0, 0)),
            ],
            out_specs=[
                pl.BlockSpec((BE, IDIM), lambda i: (i, 0)),
                pl.BlockSpec((8, IDIM), lambda i: (0, 0)),
            ],
            out_shape=[
                jax.ShapeDtypeStruct((EC, IDIM), bf),
                jax.ShapeDtypeStruct((8, IDIM), jnp.float32),
            ],
        )

    def make_pass2(c, aliased):
        in_specs = [
            pl.BlockSpec((BE2, IDIM), lambda i: (i, 0)),
            pl.BlockSpec((8 * C, IDIM), lambda i: (0, 0)),
            pl.BlockSpec((1, IDIM), lambda i: (0, 0)),
            pl.BlockSpec((1, IDIM), lambda i: (0, 0)),
            pl.BlockSpec((IDIM, DE), lambda i: (0, 0)),
            pl.BlockSpec((DE, 1), lambda i: (0, 0)),
        ]
        if aliased:
            in_specs.append(pl.BlockSpec(memory_space=pl.ANY))
        return pl.pallas_call(
            _pass2_body_aliased if aliased else _pass2_body,
            grid=(NBLK2,),
            in_specs=in_specs,
            out_specs=pl.BlockSpec((DE, BE2), lambda i, c=c: (0, i + c * NBLK2)),
            out_shape=jax.ShapeDtypeStruct((DE, E), jnp.float32),
            input_output_aliases={6: 0} if aliased else {},
        )

    z_parts, st_parts = [], []
    for c in range(C):
        gi, gj = _sc_gather(node_feats, idx_flat, c * EC)
        z_c, st_c = make_pass1(c)(gi, gj, ef_t, w1a, w1b, w1c)
        z_parts.append(z_c)
        st_parts.append(st_c)

    stats = jnp.concatenate(st_parts, axis=0)
    out = None
    for c in range(C):
        args = (z_parts[c], stats, gamma2, beta2, w2t, b22)
        if c == 0:
            out = make_pass2(0, aliased=False)(*args)
        else:
            out = make_pass2(c, aliased=True)(*args, out)
    return out.T


# speedup vs baseline: 1.0835x; 1.0835x over previous
import functools

import jax
import jax.numpy as jnp
from jax import lax
from jax.experimental import pallas as pl
from jax.experimental.pallas import tpu as pltpu
from jax.experimental.pallas import tpu_sc as plsc

N, E, D, DE = 10000, 320000, 128, 16
IDIM = 2 * D + DE

C = 5
EC = E // C

NC, NS = 2, 16
NW = NC * NS
E_PER = EC // NW
CB = 80
ITERS = E_PER // CB

BE = 3200
NBLK = EC // BE
BE2 = 12800
NBLK2 = EC // BE2


def _gather_body(coff, idx_hbm, table_hbm, out_i_hbm, out_j_hbm,
                 idx_i_v, idx_j_v, rows_i, rows_j, gsem_i, gsem_j,
                 wsem_i, wsem_j):
    wid = lax.axis_index("s") * NC + lax.axis_index("c")
    base0 = wid * E_PER

    pltpu.sync_copy(idx_hbm.at[pl.ds(coff + base0, E_PER)], idx_i_v)
    pltpu.sync_copy(idx_hbm.at[pl.ds(E + coff + base0, E_PER)], idx_j_v)

    def gather(t):
        b = t % 2
        gi = pltpu.async_copy(table_hbm.at[idx_i_v.at[pl.ds(t * CB, CB)]],
                              rows_i[b], gsem_i[b])
        gj = pltpu.async_copy(table_hbm.at[idx_j_v.at[pl.ds(t * CB, CB)]],
                              rows_j[b], gsem_j[b])
        return gi, gj

    def write(t):
        b = t % 2
        base = base0 + t * CB
        wi = pltpu.async_copy(rows_i[b], out_i_hbm.at[pl.ds(base, CB)],
                              wsem_i[b])
        wj = pltpu.async_copy(rows_j[b], out_j_hbm.at[pl.ds(base, CB)],
                              wsem_j[b])
        return wi, wj

    gd = {0: gather(0)}
    wd = {}
    for t in range(ITERS):
        if t + 1 < ITERS:
            if t - 1 >= 0:
                for c in wd.pop(t - 1):
                    c.wait()
            gd[t + 1] = gather(t + 1)
        for c in gd.pop(t):
            c.wait()
        wd[t] = write(t)
    for t in (ITERS - 2, ITERS - 1):
        for c in wd.pop(t, ()):
            c.wait()


def _sc_gather(table, idx_flat, coff):
    mesh = plsc.VectorSubcoreMesh(core_axis_name="c", subcore_axis_name="s")
    k = pl.kernel(
        functools.partial(_gather_body, coff),
        out_type=[
            jax.ShapeDtypeStruct((EC, D), jnp.float32),
            jax.ShapeDtypeStruct((EC, D), jnp.float32),
        ],
        mesh=mesh,
        scratch_types=[
            pltpu.VMEM((E_PER,), jnp.int32),
            pltpu.VMEM((E_PER,), jnp.int32),
            [pltpu.VMEM((CB, D), jnp.float32) for _ in range(2)],
            [pltpu.VMEM((CB, D), jnp.float32) for _ in range(2)],
            [pltpu.SemaphoreType.DMA for _ in range(2)],
            [pltpu.SemaphoreType.DMA for _ in range(2)],
            [pltpu.SemaphoreType.DMA for _ in range(2)],
            [pltpu.SemaphoreType.DMA for _ in range(2)],
        ],
    )
    return k(idx_flat, table)


def _pass1_body(gi_ref, gj_ref, ef_ref, w1a_ref, w1b_ref, w1c_ref,
                z_ref, stats_ref):
    step = pl.program_id(0)
    gi = gi_ref[...]
    gj = gj_ref[...]
    s = (gi + gj).astype(jnp.bfloat16)
    d = jnp.abs(gi - gj).astype(jnp.bfloat16)
    eft = ef_ref[...].astype(jnp.bfloat16)
    z = jnp.dot(s, w1a_ref[...], preferred_element_type=jnp.float32)
    z += jnp.dot(d, w1b_ref[...], preferred_element_type=jnp.float32)
    z += lax.dot_general(eft, w1c_ref[...], (((0,), (0,)), ((), ())),
                         preferred_element_type=jnp.float32)
    z_ref[...] = z.astype(z_ref.dtype)

    sums = jnp.sum(z, axis=0)
    sqs = jnp.sum(z * z, axis=0)
    acc = jnp.concatenate(
        [sums[None, :], sqs[None, :], jnp.zeros((6, IDIM), jnp.float32)], axis=0)

    @pl.when(step == 0)
    def _():
        stats_ref[...] = jnp.zeros_like(stats_ref)

    stats_ref[...] += acc


def _pass2_body_aliased(z_ref, stats_ref, gamma_ref, beta_ref, w2t_ref,
                        b2_ref, buf_ref, out_ref):
    del buf_ref
    _pass2_body(z_ref, stats_ref, gamma_ref, beta_ref, w2t_ref, b2_ref,
                out_ref)


def _pass2_body(z_ref, stats_ref, gamma_ref, beta_ref, w2t_ref, b2_ref,
                out_ref):
    st = stats_ref[...]
    tot = st[0:8, :]
    for k in range(1, C):
        tot = tot + st[8 * k:8 * (k + 1), :]
    mean = tot[0:1, :] / E
    var = tot[1:2, :] / E - mean * mean
    a = gamma_ref[...] * lax.rsqrt(var + 1e-5)
    c = beta_ref[...] - mean * a
    y = jnp.maximum(z_ref[...].astype(jnp.float32) * a + c, 0.0)
    out_ref[...] = (
        lax.dot_general(w2t_ref[...], y, (((0,), (1,)), ((), ())),
                        preferred_element_type=jnp.float32)
        + b2_ref[...])


def kernel(node_feats, edge_feats, edge_index, W1, b1, gamma, beta, W2, b2):
    del b1
    idx_flat = edge_index.reshape(2 * E)
    bf = jnp.bfloat16

    w1a = W1[:, :D].T.astype(bf)
    w1b = W1[:, D:2 * D].T.astype(bf)
    w1c = W1[:, 2 * D:].T.astype(bf)
    w2t = W2.T
    gamma2 = gamma.reshape(1, IDIM)
    beta2 = beta.reshape(1, IDIM)
    b22 = b2.reshape(DE, 1)
    ef_t = edge_feats.T

    def make_pass1(c):
        return pl.pallas_call(
            _pass1_body,
            grid=(NBLK,),
            in_specs=[
                pl.BlockSpec((BE, D), lambda i: (i, 0)),
                pl.BlockSpec((BE, D), lambda i: (i, 0)),
                pl.BlockSpec((DE, BE), lambda i, c=c: (0, i + c * NBLK)),
                pl.BlockSpec((D, IDIM), lambda i: (0, 0)),
                pl.BlockSpec((D, IDIM), lambda i: (0, 0)),
                pl.BlockSpec((DE, IDIM), lambda i: (0, 0)),
            ],
            out_specs=[
                pl.BlockSpec((BE, IDIM), lambda i: (i, 0)),
                pl.BlockSpec((8, IDIM), lambda i: (0, 0)),
            ],
            out_shape=[
                jax.ShapeDtypeStruct((EC, IDIM), bf),
                jax.ShapeDtypeStruct((8, IDIM), jnp.float32),
            ],
        )

    def make_pass2(c, aliased):
        in_specs = [
            pl.BlockSpec((BE2, IDIM), lambda i: (i, 0)),
            pl.BlockSpec((8 * C, IDIM), lambda i: (0, 0)),
            pl.BlockSpec((1, IDIM), lambda i: (0, 0)),
            pl.BlockSpec((1, IDIM), lambda i: (0, 0)),
            pl.BlockSpec((IDIM, DE), lambda i: (0, 0)),
            pl.BlockSpec((DE, 1), lambda i: (0, 0)),
        ]
        if aliased:
            in_specs.append(pl.BlockSpec(memory_space=pl.ANY))
        return pl.pallas_call(
            _pass2_body_aliased if aliased else _pass2_body,
            grid=(NBLK2,),
            in_specs=in_specs,
            out_specs=pl.BlockSpec((DE, BE2), lambda i, c=c: (0, i + c * NBLK2)),
            out_shape=jax.ShapeDtypeStruct((DE, E), jnp.float32),
            input_output_aliases={6: 0} if aliased else {},
        )

    z_parts, st_parts = [], []
    for c in range(C):
        gi, gj = _sc_gather(node_feats, idx_flat, c * EC)
        z_c, st_c = make_pass1(c)(gi, gj, ef_t, w1a, w1b, w1c)
        z_parts.append(z_c)
        st_parts.append(st_c)

    stats = jnp.concatenate(st_parts, axis=0)
    out = None
    for c in range(C):
        args = (z_parts[c], stats, gamma2, beta2, w2t, b22)
        if c == 0:
            out = make_pass2(0, aliased=False)(*args)
        else:
            out = make_pass2(c, aliased=True)(*args, out)
    return out.T


# speedup vs baseline: 1.0917x; 1.0075x over previous
import functools

import jax
import jax.numpy as jnp
from jax import lax
from jax.experimental import pallas as pl
from jax.experimental.pallas import tpu as pltpu
from jax.experimental.pallas import tpu_sc as plsc

N, E, D, DE = 10000, 320000, 128, 16
IDIM = 2 * D + DE

C = 5
EC = E // C

NC, NS = 2, 16
NW = NC * NS
E_PER = EC // NW
CB = 80
ITERS = E_PER // CB

BE = 6400
NBLK = EC // BE
BE2 = 12800
NBLK2 = EC // BE2


def _gather_body(coff, idx_hbm, table_hbm, out_i_hbm, out_j_hbm,
                 idx_i_v, idx_j_v, rows_i, rows_j, gsem_i, gsem_j,
                 wsem_i, wsem_j):
    wid = lax.axis_index("s") * NC + lax.axis_index("c")
    base0 = wid * E_PER

    pltpu.sync_copy(idx_hbm.at[pl.ds(coff + base0, E_PER)], idx_i_v)
    pltpu.sync_copy(idx_hbm.at[pl.ds(E + coff + base0, E_PER)], idx_j_v)

    def gather(t):
        b = t % 2
        gi = pltpu.async_copy(table_hbm.at[idx_i_v.at[pl.ds(t * CB, CB)]],
                              rows_i[b], gsem_i[b])
        gj = pltpu.async_copy(table_hbm.at[idx_j_v.at[pl.ds(t * CB, CB)]],
                              rows_j[b], gsem_j[b])
        return gi, gj

    def write(t):
        b = t % 2
        base = base0 + t * CB
        wi = pltpu.async_copy(rows_i[b], out_i_hbm.at[pl.ds(base, CB)],
                              wsem_i[b])
        wj = pltpu.async_copy(rows_j[b], out_j_hbm.at[pl.ds(base, CB)],
                              wsem_j[b])
        return wi, wj

    gd = {0: gather(0)}
    wd = {}
    for t in range(ITERS):
        if t + 1 < ITERS:
            if t - 1 >= 0:
                for c in wd.pop(t - 1):
                    c.wait()
            gd[t + 1] = gather(t + 1)
        for c in gd.pop(t):
            c.wait()
        wd[t] = write(t)
    for t in (ITERS - 2, ITERS - 1):
        for c in wd.pop(t, ()):
            c.wait()


def _sc_gather(table, idx_flat, coff):
    mesh = plsc.VectorSubcoreMesh(core_axis_name="c", subcore_axis_name="s")
    k = pl.kernel(
        functools.partial(_gather_body, coff),
        out_type=[
            jax.ShapeDtypeStruct((EC, D), jnp.float32),
            jax.ShapeDtypeStruct((EC, D), jnp.float32),
        ],
        mesh=mesh,
        scratch_types=[
            pltpu.VMEM((E_PER,), jnp.int32),
            pltpu.VMEM((E_PER,), jnp.int32),
            [pltpu.VMEM((CB, D), jnp.float32) for _ in range(2)],
            [pltpu.VMEM((CB, D), jnp.float32) for _ in range(2)],
            [pltpu.SemaphoreType.DMA for _ in range(2)],
            [pltpu.SemaphoreType.DMA for _ in range(2)],
            [pltpu.SemaphoreType.DMA for _ in range(2)],
            [pltpu.SemaphoreType.DMA for _ in range(2)],
        ],
    )
    return k(idx_flat, table)


def _pass1_body(gi_ref, gj_ref, ef_ref, w1a_ref, w1b_ref, w1c_ref,
                z_ref, stats_ref):
    step = pl.program_id(0)
    gi = gi_ref[...]
    gj = gj_ref[...]
    s = (gi + gj).astype(jnp.bfloat16)
    d = jnp.abs(gi - gj).astype(jnp.bfloat16)
    eft = ef_ref[...].astype(jnp.bfloat16)
    z = jnp.dot(s, w1a_ref[...], preferred_element_type=jnp.float32)
    z += jnp.dot(d, w1b_ref[...], preferred_element_type=jnp.float32)
    z += lax.dot_general(eft, w1c_ref[...], (((0,), (0,)), ((), ())),
                         preferred_element_type=jnp.float32)
    z_ref[...] = z.astype(z_ref.dtype)

    sums = jnp.sum(z, axis=0)
    sqs = jnp.sum(z * z, axis=0)
    acc = jnp.concatenate(
        [sums[None, :], sqs[None, :], jnp.zeros((6, IDIM), jnp.float32)], axis=0)

    @pl.when(step == 0)
    def _():
        stats_ref[...] = jnp.zeros_like(stats_ref)

    stats_ref[...] += acc


def _pass2_body_aliased(z_ref, stats_ref, gamma_ref, beta_ref, w2t_ref,
                        b2_ref, buf_ref, out_ref):
    del buf_ref
    _pass2_body(z_ref, stats_ref, gamma_ref, beta_ref, w2t_ref, b2_ref,
                out_ref)


def _pass2_body(z_ref, stats_ref, gamma_ref, beta_ref, w2t_ref, b2_ref,
                out_ref):
    st = stats_ref[...]
    tot = st[0:8, :]
    for k in range(1, C):
        tot = tot + st[8 * k:8 * (k + 1), :]
    mean = tot[0:1, :] / E
    var = tot[1:2, :] / E - mean * mean
    a = gamma_ref[...] * lax.rsqrt(var + 1e-5)
    c = beta_ref[...] - mean * a
    y = jnp.maximum(z_ref[...].astype(jnp.float32) * a + c, 0.0)
    out_ref[...] = (
        lax.dot_general(w2t_ref[...], y, (((0,), (1,)), ((), ())),
                        preferred_element_type=jnp.float32)
        + b2_ref[...])


def kernel(node_feats, edge_feats, edge_index, W1, b1, gamma, beta, W2, b2):
    del b1
    idx_flat = edge_index.reshape(2 * E)
    bf = jnp.bfloat16

    w1a = W1[:, :D].T.astype(bf)
    w1b = W1[:, D:2 * D].T.astype(bf)
    w1c = W1[:, 2 * D:].T.astype(bf)
    w2t = W2.T
    gamma2 = gamma.reshape(1, IDIM)
    beta2 = beta.reshape(1, IDIM)
    b22 = b2.reshape(DE, 1)
    ef_t = edge_feats.T

    def make_pass1(c):
        return pl.pallas_call(
            _pass1_body,
            grid=(NBLK,),
            in_specs=[
                pl.BlockSpec((BE, D), lambda i: (i, 0)),
                pl.BlockSpec((BE, D), lambda i: (i, 0)),
                pl.BlockSpec((DE, BE), lambda i, c=c: (0, i + c * NBLK)),
                pl.BlockSpec((D, IDIM), lambda i: (0, 0)),
                pl.BlockSpec((D, IDIM), lambda i: (0, 0)),
                pl.BlockSpec((DE, IDIM), lambda i: (0, 0)),
            ],
            out_specs=[
                pl.BlockSpec((BE, IDIM), lambda i: (i, 0)),
                pl.BlockSpec((8, IDIM), lambda i: (0, 0)),
            ],
            out_shape=[
                jax.ShapeDtypeStruct((EC, IDIM), bf),
                jax.ShapeDtypeStruct((8, IDIM), jnp.float32),
            ],
        )

    def make_pass2(c, aliased):
        in_specs = [
            pl.BlockSpec((BE2, IDIM), lambda i: (i, 0)),
            pl.BlockSpec((8 * C, IDIM), lambda i: (0, 0)),
            pl.BlockSpec((1, IDIM), lambda i: (0, 0)),
            pl.BlockSpec((1, IDIM), lambda i: (0, 0)),
            pl.BlockSpec((IDIM, DE), lambda i: (0, 0)),
            pl.BlockSpec((DE, 1), lambda i: (0, 0)),
        ]
        if aliased:
            in_specs.append(pl.BlockSpec(memory_space=pl.ANY))
        return pl.pallas_call(
            _pass2_body_aliased if aliased else _pass2_body,
            grid=(NBLK2,),
            in_specs=in_specs,
            out_specs=pl.BlockSpec((DE, BE2), lambda i, c=c: (0, i + c * NBLK2)),
            out_shape=jax.ShapeDtypeStruct((DE, E), jnp.float32),
            input_output_aliases={6: 0} if aliased else {},
        )

    z_parts, st_parts = [], []
    for c in range(C):
        gi, gj = _sc_gather(node_feats, idx_flat, c * EC)
        z_c, st_c = make_pass1(c)(gi, gj, ef_t, w1a, w1b, w1c)
        z_parts.append(z_c)
        st_parts.append(st_c)

    stats = jnp.concatenate(st_parts, axis=0)
    out = None
    for c in range(C):
        args = (z_parts[c], stats, gamma2, beta2, w2t, b22)
        if c == 0:
            out = make_pass2(0, aliased=False)(*args)
        else:
            out = make_pass2(c, aliased=True)(*args, out)
    return out.T
